# Initial kernel scaffold; baseline (speedup 1.0000x reference)
import functools

import jax
import jax.numpy as jnp
from jax import lax
from jax.experimental import pallas as pl
from jax.experimental.pallas import tpu as pltpu
from jax.experimental.pallas import tpu_sc as plsc

N = 10000
D = 128
E = 320000
NC = 2
NS = 16
NW = NC * NS
N_PAD = 10240
RPT = N_PAD // NS
EPW = E // NW
K = 80
BPW = EPW // K
HREP = 8

_MESH = plsc.VectorSubcoreMesh(core_axis_name="c", subcore_axis_name="s")


@functools.partial(
    pl.kernel,
    out_type=jax.ShapeDtypeStruct((NC * N_PAD, HREP), jnp.float32),
    mesh=_MESH,
    scratch_types=[
        pltpu.VMEM((BPW, K), jnp.int32),
        pltpu.VMEM((K, HREP), jnp.float32),
        pltpu.VMEM((RPT, HREP), jnp.float32),
        pltpu.VMEM_SHARED((N_PAD, HREP), jnp.float32),
    ],
)
def _deg_kernel(dst_hbm, ones_hbm, zeros_hbm, out_hbm, didx, ones_v, zbuf, hist):
    c = lax.axis_index("c")
    s = lax.axis_index("s")
    wid = c * NS + s
    r0 = s * RPT
    pltpu.sync_copy(zeros_hbm, zbuf)
    pltpu.sync_copy(zbuf, hist.at[pl.ds(r0, RPT)])
    pltpu.sync_copy(ones_hbm, ones_v)
    pltpu.sync_copy(dst_hbm.at[pl.ds(wid * BPW, BPW)], didx)
    plsc.subcore_barrier()

    @pl.loop(0, BPW)
    def _(blk):
        pltpu.sync_copy(ones_v, hist.at[didx.at[blk]], add=True)

    plsc.subcore_barrier()
    pltpu.sync_copy(hist.at[pl.ds(r0, RPT)],
                    out_hbm.at[pl.ds(c * N_PAD + r0, RPT)])


@functools.partial(
    pl.kernel,
    out_type=jax.ShapeDtypeStruct((NC * N_PAD, D), jnp.float32),
    mesh=_MESH,
    scratch_types=[
        pltpu.VMEM((BPW, K), jnp.int32),
        pltpu.VMEM((BPW, K), jnp.int32),
        pltpu.VMEM((K, D), jnp.float32),
        pltpu.VMEM((64, D), jnp.float32),
        pltpu.VMEM_SHARED((N_PAD, D), jnp.float32),
    ],
)
def _agg_kernel(y_hbm, src_hbm, dst_hbm, zeros_hbm, out_hbm,
                sidx, didx, rows, zbuf, acc):
    c = lax.axis_index("c")
    s = lax.axis_index("s")
    wid = c * NS + s
    r0 = s * RPT
    pltpu.sync_copy(zeros_hbm, zbuf)

    @pl.loop(0, RPT // 64)
    def _(i):
        pltpu.sync_copy(zbuf, acc.at[pl.ds(r0 + i * 64, 64)])

    pltpu.sync_copy(src_hbm.at[pl.ds(wid * BPW, BPW)], sidx)
    pltpu.sync_copy(dst_hbm.at[pl.ds(wid * BPW, BPW)], didx)
    plsc.subcore_barrier()

    @pl.loop(0, BPW)
    def _(blk):
        pltpu.sync_copy(y_hbm.at[sidx.at[blk]], rows)
        pltpu.sync_copy(rows, acc.at[didx.at[blk]], add=True)

    plsc.subcore_barrier()
    pltpu.sync_copy(acc.at[pl.ds(r0, RPT)],
                    out_hbm.at[pl.ds(c * N_PAD + r0, RPT)])


def _pre_body(deg_ref, x_ref, w_ref, dis_ref, y_ref):
    d0 = deg_ref[0:N, 0:1]
    d1 = deg_ref[N_PAD:N_PAD + N, 0:1]
    dis = lax.rsqrt(d0 + d1 + 1.0)
    dis_ref[...] = dis
    y_ref[...] = jnp.dot(x_ref[...], w_ref[...],
                         preferred_element_type=jnp.float32) * dis


_pre = pl.pallas_call(
    _pre_body,
    out_shape=(jax.ShapeDtypeStruct((N, 1), jnp.float32),
               jax.ShapeDtypeStruct((N, D), jnp.float32)),
)


def _mid_body(p_ref, y_ref, dis_ref, b_ref, w_ref, o_ref):
    dis = dis_ref[...]
    agg = p_ref[0:N, :] + p_ref[N_PAD:N_PAD + N, :] + y_ref[...]
    h = jnp.maximum(agg * dis + b_ref[...], 0.0)
    o_ref[...] = jnp.dot(h, w_ref[...],
                         preferred_element_type=jnp.float32) * dis


_mid = pl.pallas_call(
    _mid_body,
    out_shape=jax.ShapeDtypeStruct((N, D), jnp.float32),
)


def _post_body(p_ref, y_ref, dis_ref, b_ref, o_ref):
    agg = p_ref[0:N, :] + p_ref[N_PAD:N_PAD + N, :] + y_ref[...]
    o_ref[...] = agg * dis_ref[...] + b_ref[...]


_post = pl.pallas_call(
    _post_body,
    out_shape=jax.ShapeDtypeStruct((N, D), jnp.float32),
)


def kernel(x, edge_index, W1, b1, W2, b2, W3, b3):
    src = edge_index[0].reshape(E // K, K)
    dst = edge_index[1].reshape(E // K, K)
    ones_h = jnp.ones((K, HREP), jnp.float32)
    zeros_h = jnp.zeros((RPT, HREP), jnp.float32)
    zeros_rows = jnp.zeros((64, D), jnp.float32)

    deg = _deg_kernel(dst, ones_h, zeros_h)
    dis, y = _pre(deg, x, W1)
    p = _agg_kernel(y, src, dst, zeros_rows)
    y = _mid(p, y, dis, b1.reshape(1, D), W2)
    p = _agg_kernel(y, src, dst, zeros_rows)
    y = _mid(p, y, dis, b2.reshape(1, D), W3)
    p = _agg_kernel(y, src, dst, zeros_rows)
    return _post(p, y, dis, b3.reshape(1, D))


# speedup vs baseline: 11.1906x; 11.1906x over previous
import functools

import jax
import jax.numpy as jnp
from jax import lax
from jax.experimental import pallas as pl
from jax.experimental.pallas import tpu as pltpu
from jax.experimental.pallas import tpu_sc as plsc

N = 10000
D = 128
E = 320000
NC = 2
NS = 16
NW = NC * NS
N_PAD = 10240
RPT = N_PAD // NS
EPW = E // NW
K = 80
BPW = EPW // K

_MESH = plsc.VectorSubcoreMesh(core_axis_name="c", subcore_axis_name="s")
_SC_PARAMS = pltpu.CompilerParams(use_tc_tiling_on_sc=False)


@functools.partial(
    pl.kernel,
    out_type=jax.ShapeDtypeStruct((NC * N_PAD, D), jnp.float32),
    mesh=_MESH,
    compiler_params=_SC_PARAMS,
    scratch_types=[
        pltpu.VMEM((K,), jnp.int32),
        pltpu.VMEM((K, D), jnp.float32),
        pltpu.VMEM((K, D), jnp.float32),
        pltpu.VMEM_SHARED((N_PAD, D), jnp.float32),
    ],
)
def _deg_kernel(dst_hbm, ones_hbm, zeros_hbm, out_hbm,
                didx, ones_v, zbuf, hist):
    c = lax.axis_index("c")
    s = lax.axis_index("s")
    wid = c * NS + s
    r0 = s * RPT
    pltpu.sync_copy(zeros_hbm, zbuf)

    @pl.loop(0, RPT // K)
    def _(i):
        pltpu.sync_copy(zbuf, hist.at[pl.ds(r0 + i * K, K)])

    pltpu.sync_copy(ones_hbm, ones_v)
    plsc.subcore_barrier()

    @pl.loop(0, BPW)
    def _(blk):
        base = pl.multiple_of(wid * EPW + blk * K, 8)
        pltpu.sync_copy(dst_hbm.at[pl.ds(base, K)], didx)
        pltpu.sync_copy(ones_v, hist.at[didx], add=True)

    plsc.subcore_barrier()

    @pl.loop(0, RPT // K)
    def _(i):
        pltpu.sync_copy(hist.at[pl.ds(r0 + i * K, K)], zbuf)
        pltpu.sync_copy(zbuf, out_hbm.at[pl.ds(c * N_PAD + r0 + i * K, K)])


@functools.partial(
    pl.kernel,
    out_type=jax.ShapeDtypeStruct((NC * N_PAD, D), jnp.float32),
    mesh=_MESH,
    compiler_params=_SC_PARAMS,
    scratch_types=[
        pltpu.VMEM((K,), jnp.int32),
        pltpu.VMEM((K,), jnp.int32),
        pltpu.VMEM((K, D), jnp.float32),
        pltpu.VMEM_SHARED((N_PAD, D), jnp.float32),
    ],
)
def _agg_kernel(y_hbm, src_hbm, dst_hbm, zeros_hbm, out_hbm,
                sidx, didx, rows, acc):
    c = lax.axis_index("c")
    s = lax.axis_index("s")
    wid = c * NS + s
    r0 = s * RPT
    pltpu.sync_copy(zeros_hbm, rows)

    @pl.loop(0, RPT // K)
    def _(i):
        pltpu.sync_copy(rows, acc.at[pl.ds(r0 + i * K, K)])

    plsc.subcore_barrier()

    @pl.loop(0, BPW)
    def _(blk):
        base = pl.multiple_of(wid * EPW + blk * K, 8)
        pltpu.sync_copy(src_hbm.at[pl.ds(base, K)], sidx)
        pltpu.sync_copy(dst_hbm.at[pl.ds(base, K)], didx)
        pltpu.sync_copy(y_hbm.at[sidx], rows)
        pltpu.sync_copy(rows, acc.at[didx], add=True)

    plsc.subcore_barrier()

    @pl.loop(0, RPT // K)
    def _(i):
        pltpu.sync_copy(acc.at[pl.ds(r0 + i * K, K)], rows)
        pltpu.sync_copy(rows, out_hbm.at[pl.ds(c * N_PAD + r0 + i * K, K)])


def _pre_body(deg_ref, x_ref, w_ref, dis_ref, y_ref):
    d0 = deg_ref[0:N, 0:1]
    d1 = deg_ref[N_PAD:N_PAD + N, 0:1]
    dis = lax.rsqrt(d0 + d1 + 1.0)
    dis_ref[...] = dis
    y_ref[...] = jnp.dot(x_ref[...], w_ref[...],
                         preferred_element_type=jnp.float32) * dis


_pre = pl.pallas_call(
    _pre_body,
    out_shape=(jax.ShapeDtypeStruct((N, 1), jnp.float32),
               jax.ShapeDtypeStruct((N, D), jnp.float32)),
)


def _mid_body(p_ref, y_ref, dis_ref, b_ref, w_ref, o_ref):
    dis = dis_ref[...]
    agg = p_ref[0:N, :] + p_ref[N_PAD:N_PAD + N, :] + y_ref[...]
    h = jnp.maximum(agg * dis + b_ref[...], 0.0)
    o_ref[...] = jnp.dot(h, w_ref[...],
                         preferred_element_type=jnp.float32) * dis


_mid = pl.pallas_call(
    _mid_body,
    out_shape=jax.ShapeDtypeStruct((N, D), jnp.float32),
)


def _post_body(p_ref, y_ref, dis_ref, b_ref, o_ref):
    agg = p_ref[0:N, :] + p_ref[N_PAD:N_PAD + N, :] + y_ref[...]
    o_ref[...] = agg * dis_ref[...] + b_ref[...]


_post = pl.pallas_call(
    _post_body,
    out_shape=jax.ShapeDtypeStruct((N, D), jnp.float32),
)


def kernel(x, edge_index, W1, b1, W2, b2, W3, b3):
    src = edge_index[0]
    dst = edge_index[1]
    ones_h = jnp.ones((K, D), jnp.float32)
    zeros_rows = jnp.zeros((K, D), jnp.float32)

    deg = _deg_kernel(dst, ones_h, zeros_rows)
    dis, y = _pre(deg, x, W1)
    p = _agg_kernel(y, src, dst, zeros_rows)
    y = _mid(p, y, dis, b1.reshape(1, D), W2)
    p = _agg_kernel(y, src, dst, zeros_rows)
    y = _mid(p, y, dis, b2.reshape(1, D), W3)
    p = _agg_kernel(y, src, dst, zeros_rows)
    return _post(p, y, dis, b3.reshape(1, D))


# speedup vs baseline: 25.5768x; 2.2856x over previous
import functools

import jax
import jax.numpy as jnp
from jax import lax
from jax.experimental import pallas as pl
from jax.experimental.pallas import tpu as pltpu
from jax.experimental.pallas import tpu_sc as plsc

N = 10000
D = 128
E = 320000
NC = 2
NS = 16
NW = NC * NS
N_PAD = 10240
RPT = N_PAD // NS
EPW = E // NW
K = 80
BPW = EPW // K

_MESH = plsc.VectorSubcoreMesh(core_axis_name="c", subcore_axis_name="s")
_SC_PARAMS = pltpu.CompilerParams(use_tc_tiling_on_sc=False)


@functools.partial(
    pl.kernel,
    out_type=jax.ShapeDtypeStruct((NC * N_PAD, D), jnp.float32),
    mesh=_MESH,
    compiler_params=_SC_PARAMS,
    scratch_types=[
        pltpu.VMEM((K,), jnp.int32),
        pltpu.VMEM((K,), jnp.int32),
        pltpu.VMEM((K, D), jnp.float32),
        pltpu.VMEM((K, D), jnp.float32),
        pltpu.SemaphoreType.DMA,
        pltpu.SemaphoreType.DMA,
        pltpu.SemaphoreType.DMA,
        pltpu.SemaphoreType.DMA,
        pltpu.SemaphoreType.DMA,
        pltpu.VMEM_SHARED((N_PAD, D), jnp.float32),
    ],
)
def _deg_kernel(dst_hbm, ones_hbm, zeros_hbm, out_hbm,
                didx0, didx1, ones_v, zbuf,
                dsem0, dsem1, ssem0, ssem1, zsem, hist):
    c = lax.axis_index("c")
    s = lax.axis_index("s")
    wid = c * NS + s
    r0 = s * RPT
    ebase = pl.multiple_of(wid * EPW, 8)
    didx = (didx0, didx1)
    dsem = (dsem0, dsem1)
    ssem = (ssem0, ssem1)

    pltpu.sync_copy(zeros_hbm, zbuf)
    for i in range(RPT // K):
        pltpu.async_copy(zbuf, hist.at[pl.ds(r0 + i * K, K)], zsem)
    for i in range(RPT // K):
        pltpu.make_async_copy(zeros_hbm, zbuf, zsem).wait()
    pltpu.sync_copy(ones_hbm, ones_v)
    plsc.subcore_barrier()

    def fetch(t, b):
        pltpu.async_copy(dst_hbm.at[pl.ds(ebase + t * K, K)], didx[b], dsem[b])

    def dwait(b):
        pltpu.make_async_copy(dst_hbm.at[pl.ds(0, K)], didx[b], dsem[b]).wait()

    def scat(b):
        pltpu.async_copy(ones_v, hist.at[didx[b]], ssem[b], add=True)

    def swait(b):
        pltpu.make_async_copy(zeros_hbm, zbuf, ssem[b]).wait()

    fetch(0, 0)

    @pl.loop(0, BPW, step=2)
    def _(j):
        @pl.when(j + 1 < BPW)
        def _():
            @pl.when(j >= 2)
            def _():
                swait(1)
            fetch(j + 1, 1)

        dwait(0)
        scat(0)

        @pl.when(j + 1 < BPW)
        def _():
            @pl.when(j + 2 < BPW)
            def _():
                swait(0)
                fetch(j + 2, 0)
            dwait(1)
            scat(1)

    swait(0)
    swait(1)
    plsc.subcore_barrier()

    for i in range(RPT // K):
        b = i % 2
        buf = (ones_v, zbuf)[b]
        if i >= 2:
            pltpu.make_async_copy(zeros_hbm, buf, dsem[b]).wait()
        pltpu.sync_copy(hist.at[pl.ds(r0 + i * K, K)], buf)
        pltpu.async_copy(buf, out_hbm.at[pl.ds(c * N_PAD + r0 + i * K, K)],
                         dsem[b])
    pltpu.make_async_copy(zeros_hbm, ones_v, dsem0).wait()
    pltpu.make_async_copy(zeros_hbm, zbuf, dsem1).wait()


@functools.partial(
    pl.kernel,
    out_type=jax.ShapeDtypeStruct((NC * N_PAD, D), jnp.float32),
    mesh=_MESH,
    compiler_params=_SC_PARAMS,
    scratch_types=[
        pltpu.VMEM((EPW,), jnp.int32),
        pltpu.VMEM((K,), jnp.int32),
        pltpu.VMEM((K,), jnp.int32),
        pltpu.VMEM((K, D), jnp.float32),
        pltpu.VMEM((K, D), jnp.float32),
        pltpu.SemaphoreType.DMA,
        pltpu.SemaphoreType.DMA,
        pltpu.SemaphoreType.DMA,
        pltpu.SemaphoreType.DMA,
        pltpu.SemaphoreType.DMA,
        pltpu.SemaphoreType.DMA,
        pltpu.SemaphoreType.DMA,
        pltpu.VMEM_SHARED((N_PAD, D), jnp.float32),
    ],
)
def _agg_kernel(y_hbm, src_hbm, dst_hbm, zeros_hbm, out_hbm,
                sidx_all, didx0, didx1, rows0, rows1,
                gsem0, gsem1, dsem0, dsem1, ssem0, ssem1, zsem, acc):
    c = lax.axis_index("c")
    s = lax.axis_index("s")
    wid = c * NS + s
    r0 = s * RPT
    ebase = pl.multiple_of(wid * EPW, 8)
    didx = (didx0, didx1)
    rows = (rows0, rows1)
    gsem = (gsem0, gsem1)
    dsem = (dsem0, dsem1)
    ssem = (ssem0, ssem1)

    pltpu.sync_copy(src_hbm.at[pl.ds(ebase, EPW)], sidx_all)
    pltpu.sync_copy(zeros_hbm, rows0)
    for i in range(RPT // K):
        pltpu.async_copy(rows0, acc.at[pl.ds(r0 + i * K, K)], zsem)
    for i in range(RPT // K):
        pltpu.make_async_copy(zeros_hbm, rows0, zsem).wait()
    plsc.subcore_barrier()

    def fetch(t, b):
        pltpu.async_copy(dst_hbm.at[pl.ds(ebase + t * K, K)], didx[b], dsem[b])
        pltpu.async_copy(y_hbm.at[sidx_all.at[pl.ds(t * K, K)]], rows[b],
                         gsem[b])

    def gwait(b):
        pltpu.make_async_copy(zeros_hbm, rows[b], gsem[b]).wait()

    def dwait(b):
        pltpu.make_async_copy(dst_hbm.at[pl.ds(0, K)], didx[b], dsem[b]).wait()

    def scat(b):
        pltpu.async_copy(rows[b], acc.at[didx[b]], ssem[b], add=True)

    def swait(b):
        pltpu.make_async_copy(zeros_hbm, rows[b], ssem[b]).wait()

    fetch(0, 0)

    @pl.loop(0, BPW, step=2)
    def _(j):
        @pl.when(j + 1 < BPW)
        def _():
            @pl.when(j >= 2)
            def _():
                swait(1)
            fetch(j + 1, 1)

        gwait(0)
        dwait(0)
        scat(0)

        @pl.when(j + 1 < BPW)
        def _():
            @pl.when(j + 2 < BPW)
            def _():
                swait(0)
                fetch(j + 2, 0)
            gwait(1)
            dwait(1)
            scat(1)

    swait(0)
    swait(1)
    plsc.subcore_barrier()

    for i in range(RPT // K):
        b = i % 2
        if i >= 2:
            pltpu.make_async_copy(zeros_hbm, rows[b], gsem[b]).wait()
        pltpu.sync_copy(acc.at[pl.ds(r0 + i * K, K)], rows[b])
        pltpu.async_copy(rows[b], out_hbm.at[pl.ds(c * N_PAD + r0 + i * K, K)],
                         gsem[b])
    pltpu.make_async_copy(zeros_hbm, rows0, gsem0).wait()
    pltpu.make_async_copy(zeros_hbm, rows1, gsem1).wait()


def _pre_body(deg_ref, x_ref, w_ref, dis_ref, y_ref):
    d0 = deg_ref[0:N, 0:1]
    d1 = deg_ref[N_PAD:N_PAD + N, 0:1]
    dis = lax.rsqrt(d0 + d1 + 1.0)
    dis_ref[...] = dis
    y_ref[...] = jnp.dot(x_ref[...], w_ref[...],
                         preferred_element_type=jnp.float32) * dis


_pre = pl.pallas_call(
    _pre_body,
    out_shape=(jax.ShapeDtypeStruct((N, 1), jnp.float32),
               jax.ShapeDtypeStruct((N, D), jnp.float32)),
)


def _mid_body(p_ref, y_ref, dis_ref, b_ref, w_ref, o_ref):
    dis = dis_ref[...]
    agg = p_ref[0:N, :] + p_ref[N_PAD:N_PAD + N, :] + y_ref[...]
    h = jnp.maximum(agg * dis + b_ref[...], 0.0)
    o_ref[...] = jnp.dot(h, w_ref[...],
                         preferred_element_type=jnp.float32) * dis


_mid = pl.pallas_call(
    _mid_body,
    out_shape=jax.ShapeDtypeStruct((N, D), jnp.float32),
)


def _post_body(p_ref, y_ref, dis_ref, b_ref, o_ref):
    agg = p_ref[0:N, :] + p_ref[N_PAD:N_PAD + N, :] + y_ref[...]
    o_ref[...] = agg * dis_ref[...] + b_ref[...]


_post = pl.pallas_call(
    _post_body,
    out_shape=jax.ShapeDtypeStruct((N, D), jnp.float32),
)


def kernel(x, edge_index, W1, b1, W2, b2, W3, b3):
    src = edge_index[0]
    dst = edge_index[1]
    ones_h = jnp.ones((K, D), jnp.float32)
    zeros_rows = jnp.zeros((K, D), jnp.float32)

    deg = _deg_kernel(dst, ones_h, zeros_rows)
    dis, y = _pre(deg, x, W1)
    p = _agg_kernel(y, src, dst, zeros_rows)
    y = _mid(p, y, dis, b1.reshape(1, D), W2)
    p = _agg_kernel(y, src, dst, zeros_rows)
    y = _mid(p, y, dis, b2.reshape(1, D), W3)
    p = _agg_kernel(y, src, dst, zeros_rows)
    return _post(p, y, dis, b3.reshape(1, D))
